# baseline (speedup 1.0000x reference)
import functools
import math

import jax
import jax.numpy as jnp
from jax import lax
from jax.experimental import pallas as pl
from jax.experimental.pallas import tpu as pltpu
from jax.experimental.pallas import tpu_sc as plsc

_BZ = 512
_BE = 512
_NSUB = _BE // 128


def _bf16_dot_t(a, b):
    return lax.dot_general(
        a.astype(jnp.bfloat16), b.astype(jnp.bfloat16),
        (((1,), (1,)), ((), ())),
        preferred_element_type=jnp.float32)


def _project_kernel(cb_ref, w_ref, b_ref, e_ref, ebt_ref):
    cb = cb_ref[...]
    w = w_ref[...]
    e = _bf16_dot_t(cb, w) + b_ref[...]
    e_ref[...] = e
    ebt_ref[...] = e.astype(jnp.bfloat16).T


def _project(codebook, proj_w, proj_b):
    n, in_dim = codebook.shape
    cd = proj_w.shape[0]
    blk = 1024
    nb = n // blk
    e, ebt = pl.pallas_call(
        _project_kernel,
        grid=(nb,),
        in_specs=[
            pl.BlockSpec((blk, in_dim), lambda i: (i, 0)),
            pl.BlockSpec((cd, in_dim), lambda i: (0, 0)),
            pl.BlockSpec((1, cd), lambda i: (0, 0)),
        ],
        out_specs=[
            pl.BlockSpec((blk, cd), lambda i: (i, 0)),
            pl.BlockSpec((cd, blk), lambda i: (0, i)),
        ],
        out_shape=[
            jax.ShapeDtypeStruct((n, cd), jnp.float32),
            jax.ShapeDtypeStruct((cd, n), jnp.bfloat16),
        ],
    )(codebook, proj_w, proj_b.reshape(1, cd))
    return e, ebt


def _argmin_kernel(mu, recip, zb_ref, z2_ref, ebt_ref, e2_ref, idx_ref,
                   accv_ref, acci_ref):
    n = ebt_ref.shape[1]
    bz = zb_ref.shape[0]
    zb = zb_ref[...]
    z2c = z2_ref[...]
    accv_ref[...] = jnp.full((bz, 128), jnp.inf, jnp.float32)
    acci_ref[...] = jnp.zeros((bz, 128), jnp.int32)

    def chunk(j, carry):
        ebt_c = ebt_ref[:, pl.ds(j * _BE, _BE)]
        q = lax.dot_general(zb, ebt_c, (((1,), (0,)), ((), ())),
                            preferred_element_type=jnp.float32)
        av = accv_ref[...]
        ai = acci_ref[...]
        for sub in range(_NSUB):
            e2s = e2_ref[0, pl.ds(j * _BE + sub * 128, 128)]
            s = (z2c + e2s[None, :]) + q[:, sub * 128:(sub + 1) * 128]
            s = jnp.maximum(s, 0.0)
            nk = (lax.rsqrt(s) * s + mu) * recip
            take = nk < av
            av = jnp.where(take, nk, av)
            ai = jnp.where(take, j * _NSUB + sub, ai)
        accv_ref[...] = av
        acci_ref[...] = ai
        return carry

    lax.fori_loop(0, n // _BE, chunk, 0, unroll=8)
    av = accv_ref[...]
    ai = acci_ref[...]
    cmin = jnp.min(av, axis=1, keepdims=True)
    io = lax.broadcasted_iota(jnp.int32, (bz, 128), 1)
    gidx = ai * 128 + io
    cand = jnp.where(av == cmin, gidx, n)
    idx_ref[0, 0, :] = jnp.min(cand, axis=1)


def _argmin(zb, z2rep, ebt, e2, mu, recip):
    m, c = zb.shape
    n = ebt.shape[1]
    nz = m // _BZ
    out = pl.pallas_call(
        functools.partial(_argmin_kernel, mu, recip),
        grid=(nz,),
        in_specs=[
            pl.BlockSpec((_BZ, c), lambda i: (i, 0)),
            pl.BlockSpec((_BZ, 128), lambda i: (i, 0)),
            pl.BlockSpec((c, n), lambda i: (0, 0)),
            pl.BlockSpec((1, n), lambda i: (0, 0)),
        ],
        out_specs=pl.BlockSpec((1, 1, _BZ), lambda i: (i, 0, 0)),
        out_shape=jax.ShapeDtypeStruct((nz, 1, _BZ), jnp.int32),
        scratch_shapes=[
            pltpu.VMEM((_BZ, 128), jnp.float32),
            pltpu.VMEM((_BZ, 128), jnp.int32),
        ],
    )(zb, z2rep, ebt, e2)
    return out.reshape(m)


def _gather_rows(table, idx):
    n, d = table.shape
    b = idx.shape[0]
    nw = 32
    bw = b // nw
    ch = 256
    mesh = plsc.VectorSubcoreMesh(core_axis_name="c", subcore_axis_name="s")

    @functools.partial(
        pl.kernel, mesh=mesh,
        out_type=jax.ShapeDtypeStruct((b, d), jnp.float32),
        scratch_types=[
            pltpu.VMEM((ch,), jnp.int32),
            pltpu.VMEM((ch, d), jnp.float32),
            pltpu.SemaphoreType.DMA,
        ],
    )
    def k(table_hbm, idx_hbm, out_hbm, idx_v, rows_v, sem):
        wid = lax.axis_index("s") * 2 + lax.axis_index("c")
        for t in range(bw // ch):
            base = wid * bw + t * ch
            pltpu.sync_copy(idx_hbm.at[pl.ds(base, ch)], idx_v)
            pltpu.async_copy(table_hbm.at[idx_v], rows_v, sem).wait()
            pltpu.sync_copy(rows_v, out_hbm.at[pl.ds(base, ch)])

    return k(table, idx)


def kernel(input, codebook, proj_w, proj_b):
    b, h, w, c = input.shape
    z = input.reshape(-1, c)
    m = z.shape[0]
    e, ebt = _project(codebook, proj_w, proj_b)
    z2 = jnp.sum(z * z, axis=1)
    e2 = jnp.sum(e * e, axis=1)
    zb = (-2.0 * z).astype(jnp.bfloat16)
    z2rep = jnp.broadcast_to(z2[:, None], (m, 128))
    mean = math.sqrt(2) * math.exp(math.lgamma((c + 1) / 2) - math.lgamma(c / 2))
    std = math.sqrt(c - mean ** 2)
    import numpy as _np
    mu = float(_np.float32(mean))
    recip = float(_np.float32(1.0 / std))
    zidx = _argmin(zb, z2rep, ebt, e2.reshape(1, -1), mu, recip)
    quant = _gather_rows(e, zidx)
    return zidx.reshape(b, h, w), quant.reshape(b, h, w, c)


# speedup vs baseline: 1.4575x; 1.4575x over previous
import functools
import math

import jax
import jax.numpy as jnp
from jax import lax
from jax.experimental import pallas as pl
from jax.experimental.pallas import tpu as pltpu
from jax.experimental.pallas import tpu_sc as plsc

_BZ = 512
_BE = 512
_NSUB = _BE // 128


def _bf16_dot_t(a, b):
    return lax.dot_general(
        a.astype(jnp.bfloat16), b.astype(jnp.bfloat16),
        (((1,), (1,)), ((), ())),
        preferred_element_type=jnp.float32)


def _project_kernel(cb_ref, w_ref, b_ref, e_ref, ebt_ref):
    cb = cb_ref[...]
    w = w_ref[...]
    e = _bf16_dot_t(cb, w) + b_ref[...]
    e_ref[...] = e
    ebt_ref[...] = e.astype(jnp.bfloat16).T


def _project(codebook, proj_w, proj_b):
    n, in_dim = codebook.shape
    cd = proj_w.shape[0]
    blk = 1024
    nb = n // blk
    e, ebt = pl.pallas_call(
        _project_kernel,
        grid=(nb,),
        in_specs=[
            pl.BlockSpec((blk, in_dim), lambda i: (i, 0)),
            pl.BlockSpec((cd, in_dim), lambda i: (0, 0)),
            pl.BlockSpec((1, cd), lambda i: (0, 0)),
        ],
        out_specs=[
            pl.BlockSpec((blk, cd), lambda i: (i, 0)),
            pl.BlockSpec((cd, blk), lambda i: (0, i)),
        ],
        out_shape=[
            jax.ShapeDtypeStruct((n, cd), jnp.float32),
            jax.ShapeDtypeStruct((cd, n), jnp.bfloat16),
        ],
    )(codebook, proj_w, proj_b.reshape(1, cd))
    return e, ebt


def _argmin_kernel(mu, recip, zb_ref, z2_ref, ebt_ref, e2_ref, idx_ref,
                   accv_ref, acci_ref):
    n = ebt_ref.shape[1]
    bz = zb_ref.shape[0]
    zb = zb_ref[...]
    z2c = z2_ref[...]
    accv_ref[...] = jnp.full((bz, 128), jnp.inf, jnp.float32)
    acci_ref[...] = jnp.zeros((bz, 128), jnp.int32)

    def chunk(j, carry):
        ebt_c = ebt_ref[:, pl.ds(j * _BE, _BE)]
        q = lax.dot_general(zb, ebt_c, (((1,), (0,)), ((), ())),
                            preferred_element_type=jnp.float32)
        av = accv_ref[...]
        ai = acci_ref[...]
        for sub in range(_NSUB):
            e2s = e2_ref[0, pl.ds(j * _BE + sub * 128, 128)]
            s = (z2c + e2s[None, :]) + q[:, sub * 128:(sub + 1) * 128]
            s = jnp.maximum(s, 0.0)
            take = s < av
            av = jnp.where(take, s, av)
            ai = jnp.where(take, j * _NSUB + sub, ai)
        accv_ref[...] = av
        acci_ref[...] = ai
        return carry

    lax.fori_loop(0, n // _BE, chunk, 0, unroll=8)
    av = accv_ref[...]
    ai = acci_ref[...]
    kv = (lax.rsqrt(av) * av + mu) * recip
    cmin = jnp.min(kv, axis=1, keepdims=True)
    io = lax.broadcasted_iota(jnp.int32, (bz, 128), 1)
    gidx = ai * 128 + io
    cand = jnp.where(kv == cmin, gidx, n)
    idx_ref[0, 0, :] = jnp.min(cand, axis=1)


def _argmin(zb, z2rep, ebt, e2, mu, recip):
    m, c = zb.shape
    n = ebt.shape[1]
    nz = m // _BZ
    out = pl.pallas_call(
        functools.partial(_argmin_kernel, mu, recip),
        grid=(nz,),
        in_specs=[
            pl.BlockSpec((_BZ, c), lambda i: (i, 0)),
            pl.BlockSpec((_BZ, 128), lambda i: (i, 0)),
            pl.BlockSpec((c, n), lambda i: (0, 0)),
            pl.BlockSpec((1, n), lambda i: (0, 0)),
        ],
        out_specs=pl.BlockSpec((1, 1, _BZ), lambda i: (i, 0, 0)),
        out_shape=jax.ShapeDtypeStruct((nz, 1, _BZ), jnp.int32),
        scratch_shapes=[
            pltpu.VMEM((_BZ, 128), jnp.float32),
            pltpu.VMEM((_BZ, 128), jnp.int32),
        ],
    )(zb, z2rep, ebt, e2)
    return out.reshape(m)


def _gather_rows(table, idx):
    n, d = table.shape
    b = idx.shape[0]
    nw = 32
    bw = b // nw
    ch = 256
    mesh = plsc.VectorSubcoreMesh(core_axis_name="c", subcore_axis_name="s")

    @functools.partial(
        pl.kernel, mesh=mesh,
        out_type=jax.ShapeDtypeStruct((b, d), jnp.float32),
        scratch_types=[
            pltpu.VMEM((ch,), jnp.int32),
            pltpu.VMEM((ch, d), jnp.float32),
            pltpu.SemaphoreType.DMA,
        ],
    )
    def k(table_hbm, idx_hbm, out_hbm, idx_v, rows_v, sem):
        wid = lax.axis_index("s") * 2 + lax.axis_index("c")
        for t in range(bw // ch):
            base = wid * bw + t * ch
            pltpu.sync_copy(idx_hbm.at[pl.ds(base, ch)], idx_v)
            pltpu.async_copy(table_hbm.at[idx_v], rows_v, sem).wait()
            pltpu.sync_copy(rows_v, out_hbm.at[pl.ds(base, ch)])

    return k(table, idx)


def kernel(input, codebook, proj_w, proj_b):
    b, h, w, c = input.shape
    z = input.reshape(-1, c)
    m = z.shape[0]
    e, ebt = _project(codebook, proj_w, proj_b)
    z2 = jnp.sum(z * z, axis=1)
    e2 = jnp.sum(e * e, axis=1)
    zb = (-2.0 * z).astype(jnp.bfloat16)
    z2rep = jnp.broadcast_to(z2[:, None], (m, 128))
    mean = math.sqrt(2) * math.exp(math.lgamma((c + 1) / 2) - math.lgamma(c / 2))
    std = math.sqrt(c - mean ** 2)
    import numpy as _np
    mu = float(_np.float32(mean))
    recip = float(_np.float32(1.0 / std))
    zidx = _argmin(zb, z2rep, ebt, e2.reshape(1, -1), mu, recip)
    quant = _gather_rows(e, zidx)
    return zidx.reshape(b, h, w), quant.reshape(b, h, w, c)
